# baseline (speedup 1.0000x reference)
import functools

import jax
import jax.numpy as jnp
from jax import lax
from jax.experimental import pallas as pl
from jax.experimental.pallas import tpu as pltpu
from jax.experimental.pallas import tpu_sc as plsc

N_ROWS = 32000
N_CLS = 91
N_WORKERS = 32
ROWS_PER_W = N_ROWS // N_WORKERS
PAD_ROWS = 1024
BLK = 128
ROWS_PER_PASS = 512
N_PASS = PAD_ROWS // ROWS_PER_PASS
CHUNK = 128


def _worker(tci_hbm, tb_hbm, table_hbm, out_hbm,
            tci_v, idx_v, rows_v, tb_v, part_v, sem):
    wid = lax.axis_index("c") * 16 + lax.axis_index("s")
    row0 = wid * ROWS_PER_W

    pltpu.sync_copy(tci_hbm.at[pl.ds(row0, ROWS_PER_W)], tci_v)
    pltpu.sync_copy(tb_hbm.at[pl.ds(row0 * 4, ROWS_PER_W * 4)], tb_v)

    iota = lax.iota(jnp.int32, 16)

    def idx_body(j, carry):
        r = jnp.minimum(j * 16 + iota, ROWS_PER_W - 1)
        t = plsc.load_gather(tci_v, [r])
        cls = jnp.clip(t, 0, N_CLS - 1)
        bbox_row = (row0 + r) * N_CLS + cls
        idx_v[pl.ds(pl.multiple_of(j * 16, 16), 16)] = bbox_row >> 5
        return carry

    lax.fori_loop(0, PAD_ROWS // 16, idx_body, 0)

    zero = jnp.zeros((16,), jnp.float32)
    acc = zero
    cnt = zero

    for p in range(N_PASS):
        copies = [
            pltpu.async_copy(
                table_hbm.at[idx_v.at[pl.ds(p * ROWS_PER_PASS + k * CHUNK, CHUNK)]],
                rows_v.at[pl.ds(k * CHUNK, CHUNK)],
                sem,
            )
            for k in range(ROWS_PER_PASS // CHUNK)
        ]
        for cp in copies:
            cp.wait()

        lo_e = p * ROWS_PER_PASS * 4
        hi_e = min((p + 1) * ROWS_PER_PASS, ROWS_PER_W) * 4

        def body(j, carry, p=p):
            a, n = carry
            e = j * 16 + iota
            r = e >> 2
            t = plsc.load_gather(tci_v, [r])
            m = t > 0
            cls = jnp.clip(t, 0, N_CLS - 1)
            bbox_row = (row0 + r) * N_CLS + cls
            col = (bbox_row & 31) * 4 + (e & 3)
            sel = plsc.load_gather(rows_v, [r - p * ROWS_PER_PASS, col])
            tb = tb_v[pl.ds(pl.multiple_of(j * 16, 16), 16)]
            d = jnp.abs(tb - sel)
            l = jnp.where(d < 1.0, 0.5 * d * d, d - 0.5)
            a = a + jnp.where(m, l, 0.0)
            n = n + jnp.where(m, 1.0, 0.0)
            return a, n

        acc, cnt = lax.fori_loop(lo_e // 16, hi_e // 16, body, (acc, cnt))

    part_v[0, :] = acc
    part_v[1, :] = cnt
    pltpu.sync_copy(part_v, out_hbm.at[wid])


@jax.jit
def _sc_loss(tci, tb, table):
    mesh = plsc.VectorSubcoreMesh(core_axis_name="c", subcore_axis_name="s")
    run = functools.partial(
        pl.kernel,
        mesh=mesh,
        compiler_params=pltpu.CompilerParams(needs_layout_passes=False),
        out_type=jax.ShapeDtypeStruct((N_WORKERS, 2, 16), jnp.float32),
        scratch_types=[
            pltpu.VMEM((ROWS_PER_W,), jnp.int32),
            pltpu.VMEM((PAD_ROWS,), jnp.int32),
            pltpu.VMEM((ROWS_PER_PASS, BLK), jnp.float32),
            pltpu.VMEM((ROWS_PER_W * 4,), jnp.float32),
            pltpu.VMEM((2, 16), jnp.float32),
            pltpu.SemaphoreType.DMA,
        ],
    )(_worker)
    return run(tci, tb, table)


def kernel(target_bbox, target_class_ids, pred_bbox):
    tci = target_class_ids.reshape(-1).astype(jnp.int32)
    tb = target_bbox.reshape(-1)
    table = pred_bbox.reshape(-1, BLK)
    parts = _sc_loss(tci, tb, table)
    total = parts[:, 0, :].sum()
    count = parts[:, 1, :].sum()
    return total / count


# speedup vs baseline: 33.9170x; 33.9170x over previous
import functools

import jax
import jax.numpy as jnp
from jax import lax
from jax.experimental import pallas as pl
from jax.experimental.pallas import tpu as pltpu
from jax.experimental.pallas import tpu_sc as plsc

N_CLS = 91
N_WORKERS = 32
ROWS_PER_W = 1000
N_ELEM = ROWS_PER_W * 4
PAD_SLOTS = 4096
BLK = 16


def _worker(tci_hbm, tb_hbm, table_hbm, out_hbm,
            tci_v, idx_v, rows_v, tb_v, part_v, sem):
    wid = lax.axis_index("c") * 16 + lax.axis_index("s")
    row0 = wid * ROWS_PER_W

    pltpu.sync_copy(tci_hbm.at[pl.ds(row0, ROWS_PER_W)], tci_v)
    pltpu.sync_copy(tb_hbm.at[pl.ds(row0 * 4, N_ELEM)], tb_v)

    iota = lax.iota(jnp.int32, 16)
    base = wid * (N_CLS * 4 * ROWS_PER_W)

    def flat_of(slot):
        r = jnp.minimum(slot >> 2, ROWS_PER_W - 1)
        k = slot & 3
        t = plsc.load_gather(tci_v, [r])
        cls = jnp.clip(t, 0, N_CLS - 1)
        return base + cls * (4 * ROWS_PER_W) + k * ROWS_PER_W + r

    def idx_body(j, carry):
        s = j * 16 + iota
        idx_v[pl.ds(pl.multiple_of(j * 16, 16), 16)] = flat_of(s) >> 4
        return carry

    lax.fori_loop(0, PAD_SLOTS // 16, idx_body, 0)

    pltpu.async_copy(table_hbm.at[idx_v], rows_v, sem).wait()

    zero = jnp.zeros((16,), jnp.float32)

    def body(j, carry):
        a, n = carry
        e = j * 16 + iota
        r = e >> 2
        t = plsc.load_gather(tci_v, [r])
        m = t > 0
        sel = plsc.load_gather(rows_v, [e, flat_of(e) & (BLK - 1)])
        tb = tb_v[pl.ds(pl.multiple_of(j * 16, 16), 16)]
        d = jnp.abs(tb - sel)
        l = jnp.where(d < 1.0, 0.5 * d * d, d - 0.5)
        a = a + jnp.where(m, l, 0.0)
        n = n + jnp.where(m, 1.0, 0.0)
        return a, n

    acc, cnt = lax.fori_loop(0, N_ELEM // 16, body, (zero, zero))

    part_v[0, :] = acc
    part_v[1, :] = cnt
    pltpu.sync_copy(part_v, out_hbm.at[wid])


@jax.jit
def _sc_loss(tci, tb, table):
    mesh = plsc.VectorSubcoreMesh(core_axis_name="c", subcore_axis_name="s")
    run = functools.partial(
        pl.kernel,
        mesh=mesh,
        compiler_params=pltpu.CompilerParams(
            needs_layout_passes=False, use_tc_tiling_on_sc=False
        ),
        out_type=jax.ShapeDtypeStruct((N_WORKERS, 2, 16), jnp.float32),
        scratch_types=[
            pltpu.VMEM((ROWS_PER_W,), jnp.int32),
            pltpu.VMEM((PAD_SLOTS,), jnp.int32),
            pltpu.VMEM((PAD_SLOTS, BLK), jnp.float32),
            pltpu.VMEM((N_ELEM,), jnp.float32),
            pltpu.VMEM((2, 16), jnp.float32),
            pltpu.SemaphoreType.DMA,
        ],
    )(_worker)
    return run(tci, tb, table)


def kernel(target_bbox, target_class_ids, pred_bbox):
    tci = target_class_ids.reshape(-1).astype(jnp.int32)
    tb = target_bbox.reshape(-1)
    table = jnp.transpose(pred_bbox, (0, 2, 3, 1)).reshape(-1, BLK)
    parts = _sc_loss(tci, tb, table)
    total = parts[:, 0, :].sum()
    count = parts[:, 1, :].sum()
    return total / count


# speedup vs baseline: 40.9539x; 1.2075x over previous
import functools

import jax
import jax.numpy as jnp
from jax import lax
from jax.experimental import pallas as pl
from jax.experimental.pallas import tpu as pltpu
from jax.experimental.pallas import tpu_sc as plsc

N_CLS = 91
N_WORKERS = 32
ROWS_PER_W = 1000
N_ELEM = ROWS_PER_W * 4
PAD_SLOTS = 4096
BLK = 16
RPAD = 1024


def _worker(tci_hbm, tb_hbm, table_hbm, out_hbm,
            tci_v, idx_v, rows_v, tb_v, part_v, sem):
    wid = lax.axis_index("c") * 16 + lax.axis_index("s")
    row0 = wid * ROWS_PER_W

    pltpu.sync_copy(tci_hbm.at[pl.ds(row0, ROWS_PER_W)], tci_v)
    for k in range(4):
        pltpu.sync_copy(
            tb_hbm.at[pl.ds((wid * 4 + k) * ROWS_PER_W, ROWS_PER_W)],
            tb_v.at[pl.ds(k * RPAD, ROWS_PER_W)],
        )

    iota = lax.iota(jnp.int32, 16)
    base = wid * (N_CLS * 4 * ROWS_PER_W)

    def flat_of(r, k):
        t = plsc.load_gather(tci_v, [r])
        cls = jnp.clip(t, 0, N_CLS - 1)
        return base + cls * (4 * ROWS_PER_W) + k * ROWS_PER_W + r, t

    def idx_body(j, carry):
        s = j * 16 + iota
        f, _ = flat_of(jnp.minimum(s >> 2, ROWS_PER_W - 1), s & 3)
        idx_v[pl.ds(pl.multiple_of(j * 16, 16), 16)] = f >> 4
        return carry

    lax.fori_loop(0, PAD_SLOTS // 16, idx_body, 0)

    pltpu.async_copy(table_hbm.at[idx_v], rows_v, sem).wait()

    zero = jnp.zeros((16,), jnp.float32)

    def body(j, carry):
        a, n = carry
        e = j * 16 + iota
        k = e >> 10
        r = e & (RPAD - 1)
        valid = r < ROWS_PER_W
        rc = jnp.minimum(r, ROWS_PER_W - 1)
        f, t = flat_of(rc, k)
        m = jnp.logical_and(t > 0, valid)
        sel = plsc.load_gather(rows_v, [rc * 4 + k, f & (BLK - 1)])
        tb = tb_v[pl.ds(pl.multiple_of(j * 16, 16), 16)]
        d = jnp.abs(tb - sel)
        l = jnp.where(d < 1.0, 0.5 * d * d, d - 0.5)
        a = a + jnp.where(m, l, 0.0)
        n = n + jnp.where(m, 1.0, 0.0)
        return a, n

    acc, cnt = lax.fori_loop(0, (4 * RPAD) // 16, body, (zero, zero))

    part_v[0, :] = acc
    part_v[1, :] = cnt
    pltpu.sync_copy(part_v, out_hbm.at[wid])


@jax.jit
def _sc_loss(tci, tb, table):
    mesh = plsc.VectorSubcoreMesh(core_axis_name="c", subcore_axis_name="s")
    run = functools.partial(
        pl.kernel,
        mesh=mesh,
        compiler_params=pltpu.CompilerParams(
            needs_layout_passes=False, use_tc_tiling_on_sc=False
        ),
        out_type=jax.ShapeDtypeStruct((N_WORKERS, 2, 16), jnp.float32),
        scratch_types=[
            pltpu.VMEM((ROWS_PER_W,), jnp.int32),
            pltpu.VMEM((PAD_SLOTS,), jnp.int32),
            pltpu.VMEM((PAD_SLOTS, BLK), jnp.float32),
            pltpu.VMEM((4 * RPAD,), jnp.float32),
            pltpu.VMEM((2, 16), jnp.float32),
            pltpu.SemaphoreType.DMA,
        ],
    )(_worker)
    return run(tci, tb, table)


def kernel(target_bbox, target_class_ids, pred_bbox):
    tci = target_class_ids.reshape(-1).astype(jnp.int32)
    tb = jnp.transpose(target_bbox, (0, 2, 1)).reshape(-1)
    table = jnp.transpose(pred_bbox, (0, 2, 3, 1)).reshape(-1, BLK)
    parts = _sc_loss(tci, tb, table)
    total = parts[:, 0, :].sum()
    count = parts[:, 1, :].sum()
    return total / count
